# Initial kernel scaffold; baseline (speedup 1.0000x reference)
import jax
import jax.numpy as jnp
from jax.experimental import pallas as pl


def kernel(x, W1, bn1_g, bn1_b, bn1_m, bn1_v, Wr, br, We, bne_g, bne_b, bne_m, bne_v, W3, bn3_g, bn3_b, bn3_m, bn3_v):
    raise NotImplementedError("write your pallas kernel here")


# speedup vs baseline: 1.9934x; 1.9934x over previous
import jax
import jax.numpy as jnp
from jax.experimental import pallas as pl

_E = 4
_K = 2
_EPS = 1e-3


def _silu(v):
    return v * jax.nn.sigmoid(v)


def _fused_kernel(x_ref, w1_ref, s1_ref, t1_ref, wr_ref, br_ref,
                  ce_ref, se_ref, te_ref, w3_ref, s3_ref, t3_ref, o_ref):
    xb = x_ref[0]

    h = jnp.dot(w1_ref[...], xb, preferred_element_type=jnp.float32)
    h = _silu(h * s1_ref[...] + t1_ref[...])

    logits = jnp.dot(wr_ref[...], h, preferred_element_type=jnp.float32) + br_ref[...]

    rows = [logits[e:e + 1, :] for e in range(_E)]
    m = jnp.maximum(jnp.maximum(rows[0], rows[1]), jnp.maximum(rows[2], rows[3]))
    ws = []
    for e in range(_E):
        rank = jnp.zeros_like(rows[e])
        for j in range(_E):
            if j == e:
                continue
            beats = (rows[j] > rows[e]) if j > e else (rows[j] >= rows[e])
            rank = rank + beats.astype(jnp.float32)
        sel = rank < float(_K)
        ws.append(jnp.where(sel, jnp.exp(rows[e] - m), 0.0))
    denom = ws[0] + ws[1] + ws[2] + ws[3]
    inv = 1.0 / denom

    acc = jnp.zeros_like(h)
    for e in range(_E):
        y = jnp.dot(ce_ref[e], h, preferred_element_type=jnp.float32)
        y = _silu(y * se_ref[e] + te_ref[e])
        acc = acc + (ws[e] * inv) * y

    y3 = jnp.dot(w3_ref[...], acc, preferred_element_type=jnp.float32)
    o_ref[0] = _silu(y3 * s3_ref[...] + t3_ref[...]) + xb


def kernel(x, W1, bn1_g, bn1_b, bn1_m, bn1_v, Wr, br, We,
           bne_g, bne_b, bne_m, bne_v, W3, bn3_g, bn3_b, bn3_m, bn3_v):
    B, C1, H, W = x.shape
    Ch = W1.shape[0]
    HW = H * W
    T = 2048
    x3 = x.reshape(B, C1, HW)

    s1 = (bn1_g / jnp.sqrt(bn1_v + _EPS)).reshape(Ch, 1)
    t1 = (bn1_b - bn1_m * bn1_g / jnp.sqrt(bn1_v + _EPS)).reshape(Ch, 1)
    se = (bne_g / jnp.sqrt(bne_v + _EPS)).reshape(_E, Ch, 1)
    te = (bne_b - bne_m * bne_g / jnp.sqrt(bne_v + _EPS)).reshape(_E, Ch, 1)
    s3 = (bn3_g / jnp.sqrt(bn3_v + _EPS)).reshape(C1, 1)
    t3 = (bn3_b - bn3_m * bn3_g / jnp.sqrt(bn3_v + _EPS)).reshape(C1, 1)
    ce = We[:, :, :, 1, 1]
    brc = br.reshape(_E, 1)

    grid = (B, HW // T)
    full = lambda *s: pl.BlockSpec(s, lambda b, t: (0,) * len(s))
    out = pl.pallas_call(
        _fused_kernel,
        grid=grid,
        in_specs=[
            pl.BlockSpec((1, C1, T), lambda b, t: (b, 0, t)),
            full(Ch, C1), full(Ch, 1), full(Ch, 1),
            full(_E, Ch), full(_E, 1),
            full(_E, Ch, Ch), full(_E, Ch, 1), full(_E, Ch, 1),
            full(C1, Ch), full(C1, 1), full(C1, 1),
        ],
        out_specs=pl.BlockSpec((1, C1, T), lambda b, t: (b, 0, t)),
        out_shape=jax.ShapeDtypeStruct((B, C1, HW), jnp.float32),
    )(x3, W1, s1, t1, Wr, brc, ce, se, te, W3, s3, t3)
    return out.reshape(B, C1, H, W)


# speedup vs baseline: 2.0164x; 1.0116x over previous
import jax
import jax.numpy as jnp
from jax.experimental import pallas as pl

_E = 4
_K = 2
_EPS = 1e-3


def _silu(v):
    h = 0.5 * v
    return h + h * jnp.tanh(h)


def _fused_kernel(x_ref, w1_ref, s1_ref, t1_ref, wr_ref, br_ref,
                  ce_ref, se_ref, te_ref, w3_ref, s3_ref, t3_ref, o_ref):
    xb = x_ref[0]

    h = jnp.dot(w1_ref[...], xb, preferred_element_type=jnp.float32)
    h = _silu(h * s1_ref[...] + t1_ref[...])

    logits = jnp.dot(wr_ref[...], h, preferred_element_type=jnp.float32) + br_ref[...]

    rows = [logits[e:e + 1, :] for e in range(_E)]
    m = jnp.maximum(jnp.maximum(rows[0], rows[1]), jnp.maximum(rows[2], rows[3]))
    ws = []
    for e in range(_E):
        rank = jnp.zeros_like(rows[e])
        for j in range(_E):
            if j == e:
                continue
            beats = (rows[j] > rows[e]) if j > e else (rows[j] >= rows[e])
            rank = rank + beats.astype(jnp.float32)
        sel = rank < float(_K)
        ws.append(jnp.where(sel, jnp.exp(rows[e] - m), 0.0))
    denom = ws[0] + ws[1] + ws[2] + ws[3]
    inv = 1.0 / denom

    acc = jnp.zeros_like(h)
    for e in range(_E):
        y = jnp.dot(ce_ref[e], h, preferred_element_type=jnp.float32)
        y = _silu(y * se_ref[e] + te_ref[e])
        acc = acc + (ws[e] * inv) * y

    y3 = jnp.dot(w3_ref[...], acc, preferred_element_type=jnp.float32)
    o_ref[0] = _silu(y3 * s3_ref[...] + t3_ref[...]) + xb


def kernel(x, W1, bn1_g, bn1_b, bn1_m, bn1_v, Wr, br, We,
           bne_g, bne_b, bne_m, bne_v, W3, bn3_g, bn3_b, bn3_m, bn3_v):
    B, C1, H, W = x.shape
    Ch = W1.shape[0]
    HW = H * W
    T = 2048
    x3 = x.reshape(B, C1, HW)

    s1 = (bn1_g / jnp.sqrt(bn1_v + _EPS)).reshape(Ch, 1)
    t1 = (bn1_b - bn1_m * bn1_g / jnp.sqrt(bn1_v + _EPS)).reshape(Ch, 1)
    se = (bne_g / jnp.sqrt(bne_v + _EPS)).reshape(_E, Ch, 1)
    te = (bne_b - bne_m * bne_g / jnp.sqrt(bne_v + _EPS)).reshape(_E, Ch, 1)
    s3 = (bn3_g / jnp.sqrt(bn3_v + _EPS)).reshape(C1, 1)
    t3 = (bn3_b - bn3_m * bn3_g / jnp.sqrt(bn3_v + _EPS)).reshape(C1, 1)
    ce = We[:, :, :, 1, 1]
    brc = br.reshape(_E, 1)

    grid = (B, HW // T)
    full = lambda *s: pl.BlockSpec(s, lambda b, t: (0,) * len(s))
    out = pl.pallas_call(
        _fused_kernel,
        grid=grid,
        in_specs=[
            pl.BlockSpec((1, C1, T), lambda b, t: (b, 0, t)),
            full(Ch, C1), full(Ch, 1), full(Ch, 1),
            full(_E, Ch), full(_E, 1),
            full(_E, Ch, Ch), full(_E, Ch, 1), full(_E, Ch, 1),
            full(C1, Ch), full(C1, 1), full(C1, 1),
        ],
        out_specs=pl.BlockSpec((1, C1, T), lambda b, t: (b, 0, t)),
        out_shape=jax.ShapeDtypeStruct((B, C1, HW), jnp.float32),
    )(x3, W1, s1, t1, Wr, brc, ce, se, te, W3, s3, t3)
    return out.reshape(B, C1, H, W)


# speedup vs baseline: 2.0656x; 1.0244x over previous
import jax
import jax.numpy as jnp
from jax.experimental import pallas as pl

_E = 4
_K = 2
_EPS = 1e-3


def _silu_half(z):
    return z + z * jnp.tanh(z)


def _fused_kernel(x_ref, w1_ref, t1_ref, wr_ref, br_ref,
                  ce_ref, te_ref, w3_ref, t3_ref, o_ref):
    xb = x_ref[0]

    h = _silu_half(jnp.dot(w1_ref[...], xb, preferred_element_type=jnp.float32)
                   + t1_ref[...])

    logits = jnp.dot(wr_ref[...], h, preferred_element_type=jnp.float32) + br_ref[...]

    rows = [logits[e:e + 1, :] for e in range(_E)]
    m = jnp.maximum(jnp.maximum(rows[0], rows[1]), jnp.maximum(rows[2], rows[3]))
    ws = []
    for e in range(_E):
        rank = jnp.zeros_like(rows[e])
        for j in range(_E):
            if j == e:
                continue
            beats = (rows[j] > rows[e]) if j > e else (rows[j] >= rows[e])
            rank = rank + beats.astype(jnp.float32)
        sel = rank < float(_K)
        ws.append(jnp.where(sel, jnp.exp(rows[e] - m), 0.0))
    denom = ws[0] + ws[1] + ws[2] + ws[3]
    inv = 1.0 / denom

    y = _silu_half(jnp.dot(ce_ref[...], h, preferred_element_type=jnp.float32)
                   + te_ref[...])
    Ch = h.shape[0]
    acc = (ws[0] * inv) * y[0 * Ch:1 * Ch]
    for e in range(1, _E):
        acc = acc + (ws[e] * inv) * y[e * Ch:(e + 1) * Ch]

    o_ref[0] = _silu_half(jnp.dot(w3_ref[...], acc, preferred_element_type=jnp.float32)
                          + t3_ref[...]) + xb


def kernel(x, W1, bn1_g, bn1_b, bn1_m, bn1_v, Wr, br, We,
           bne_g, bne_b, bne_m, bne_v, W3, bn3_g, bn3_b, bn3_m, bn3_v):
    B, C1, H, W = x.shape
    Ch = W1.shape[0]
    HW = H * W
    T = 2048
    x3 = x.reshape(B, C1, HW)

    s1 = 0.5 * bn1_g / jnp.sqrt(bn1_v + _EPS)
    W1f = W1 * s1[:, None]
    t1 = (0.5 * bn1_b - bn1_m * s1).reshape(Ch, 1)
    se = 0.5 * bne_g / jnp.sqrt(bne_v + _EPS)
    cef = (We[:, :, :, 1, 1] * se[:, :, None]).reshape(_E * Ch, Ch)
    tef = (0.5 * bne_b - bne_m * se).reshape(_E * Ch, 1)
    s3 = 0.5 * bn3_g / jnp.sqrt(bn3_v + _EPS)
    W3f = W3 * s3[:, None]
    t3 = (0.5 * bn3_b - bn3_m * s3).reshape(C1, 1)
    Wr8 = jnp.zeros((8, Ch), jnp.float32).at[:_E].set(Wr)
    br8 = jnp.zeros((8, 1), jnp.float32).at[:_E, 0].set(br)

    grid = (B, HW // T)
    full = lambda *s: pl.BlockSpec(s, lambda b, t: (0,) * len(s))
    out = pl.pallas_call(
        _fused_kernel,
        grid=grid,
        in_specs=[
            pl.BlockSpec((1, C1, T), lambda b, t: (b, 0, t)),
            full(Ch, C1), full(Ch, 1),
            full(8, Ch), full(8, 1),
            full(_E * Ch, Ch), full(_E * Ch, 1),
            full(C1, Ch), full(C1, 1),
        ],
        out_specs=pl.BlockSpec((1, C1, T), lambda b, t: (b, 0, t)),
        out_shape=jax.ShapeDtypeStruct((B, C1, HW), jnp.float32),
    )(x3, W1f, t1, Wr8, br8, cef, tef, W3f, t3)
    return out.reshape(B, C1, H, W)


# speedup vs baseline: 4.1517x; 2.0099x over previous
import jax
import jax.numpy as jnp
from jax.experimental import pallas as pl

_E = 4
_K = 2
_EPS = 1e-3


def _silu_half(z):
    return z + z * jnp.tanh(z)


def _fused_kernel(x_ref, w1_ref, t1_ref, wr_ref, br_ref,
                  ce_ref, te_ref, w3_ref, t3_ref, o_ref):
    xb3 = x_ref[...]
    C1, Hb, W = xb3.shape
    xb = xb3.reshape(C1, Hb * W)

    h = _silu_half(jnp.dot(w1_ref[...], xb, preferred_element_type=jnp.float32)
                   + t1_ref[...])

    logits = jnp.dot(wr_ref[...], h, preferred_element_type=jnp.float32) + br_ref[...]

    rows = [logits[e:e + 1, :] for e in range(_E)]
    m = jnp.maximum(jnp.maximum(rows[0], rows[1]), jnp.maximum(rows[2], rows[3]))
    ws = []
    for e in range(_E):
        rank = jnp.zeros_like(rows[e])
        for j in range(_E):
            if j == e:
                continue
            beats = (rows[j] > rows[e]) if j > e else (rows[j] >= rows[e])
            rank = rank + beats.astype(jnp.float32)
        sel = rank < float(_K)
        ws.append(jnp.where(sel, jnp.exp(rows[e] - m), 0.0))
    denom = ws[0] + ws[1] + ws[2] + ws[3]
    inv = 1.0 / denom

    y = _silu_half(jnp.dot(ce_ref[...], h, preferred_element_type=jnp.float32)
                   + te_ref[...])
    Ch = h.shape[0]
    acc = (ws[0] * inv) * y[0 * Ch:1 * Ch]
    for e in range(1, _E):
        acc = acc + (ws[e] * inv) * y[e * Ch:(e + 1) * Ch]

    res = _silu_half(jnp.dot(w3_ref[...], acc, preferred_element_type=jnp.float32)
                     + t3_ref[...]) + xb
    o_ref[...] = res.reshape(C1, Hb, W)


def kernel(x, W1, bn1_g, bn1_b, bn1_m, bn1_v, Wr, br, We,
           bne_g, bne_b, bne_m, bne_v, W3, bn3_g, bn3_b, bn3_m, bn3_v):
    B, C1, H, W = x.shape
    Ch = W1.shape[0]
    Hb = 16
    x3 = x.reshape(B * C1, H, W)

    s1 = 0.5 * bn1_g / jnp.sqrt(bn1_v + _EPS)
    W1f = W1 * s1[:, None]
    t1 = (0.5 * bn1_b - bn1_m * s1).reshape(Ch, 1)
    se = 0.5 * bne_g / jnp.sqrt(bne_v + _EPS)
    cef = (We[:, :, :, 1, 1] * se[:, :, None]).reshape(_E * Ch, Ch)
    tef = (0.5 * bne_b - bne_m * se).reshape(_E * Ch, 1)
    s3 = 0.5 * bn3_g / jnp.sqrt(bn3_v + _EPS)
    W3f = W3 * s3[:, None]
    t3 = (0.5 * bn3_b - bn3_m * s3).reshape(C1, 1)
    Wr8 = jnp.zeros((8, Ch), jnp.float32).at[:_E].set(Wr)
    br8 = jnp.zeros((8, 1), jnp.float32).at[:_E, 0].set(br)

    grid = (B, H // Hb)
    full = lambda *s: pl.BlockSpec(s, lambda b, t: (0,) * len(s))
    out = pl.pallas_call(
        _fused_kernel,
        grid=grid,
        in_specs=[
            pl.BlockSpec((C1, Hb, W), lambda b, t: (b, t, 0)),
            full(Ch, C1), full(Ch, 1),
            full(8, Ch), full(8, 1),
            full(_E * Ch, Ch), full(_E * Ch, 1),
            full(C1, Ch), full(C1, 1),
        ],
        out_specs=pl.BlockSpec((C1, Hb, W), lambda b, t: (b, t, 0)),
        out_shape=jax.ShapeDtypeStruct((B * C1, H, W), jnp.float32),
    )(x3, W1f, t1, Wr8, br8, cef, tef, W3f, t3)
    return out.reshape(B, C1, H, W)
